# baseline (speedup 1.0000x reference)
import jax
import jax.numpy as jnp
from jax.experimental import pallas as pl
from jax.experimental.pallas import tpu as pltpu

_DIM = 128
_HALF = 64
_BLOCK = 1024


def _coupling_kernel(x_ref, wa_ref, wb_ref, ba_ref, bb_ref, out_ref, jac_ref):
    x = x_ref[:, :]
    u = jnp.dot(x, wa_ref[:, :], preferred_element_type=jnp.float32) + ba_ref[0, :]
    sf = jnp.tanh(u)
    t2 = jnp.dot(x, wb_ref[:, :], preferred_element_type=jnp.float32) + bb_ref[0, :]
    out_ref[:, :] = x * jnp.exp(sf) + t2
    ones_col = jnp.ones((_DIM, 1), dtype=jnp.float32)
    jac_ref[:, :] = jnp.dot(sf, ones_col, preferred_element_type=jnp.float32)


def kernel(samples, W_s, b_s, W_t, b_t):
    n = samples.shape[0]
    zpad = jnp.zeros((_DIM, _DIM), dtype=jnp.float32)
    wa = zpad.at[_HALF:, :_HALF].set(W_s)
    wb = zpad.at[_HALF:, :_HALF].set(W_t)
    zvec = jnp.zeros((_HALF,), dtype=jnp.float32)
    ba = jnp.concatenate([b_s, zvec]).reshape(1, _DIM)
    bb = jnp.concatenate([b_t, zvec]).reshape(1, _DIM)
    grid = n // _BLOCK
    res, jac = pl.pallas_call(
        _coupling_kernel,
        grid=(grid,),
        in_specs=[
            pl.BlockSpec((_BLOCK, _DIM), lambda i: (i, 0)),
            pl.BlockSpec((_DIM, _DIM), lambda i: (0, 0)),
            pl.BlockSpec((_DIM, _DIM), lambda i: (0, 0)),
            pl.BlockSpec((1, _DIM), lambda i: (0, 0)),
            pl.BlockSpec((1, _DIM), lambda i: (0, 0)),
        ],
        out_specs=[
            pl.BlockSpec((_BLOCK, _DIM), lambda i: (i, 0)),
            pl.BlockSpec((_BLOCK, 1), lambda i: (i, 0)),
        ],
        out_shape=[
            jax.ShapeDtypeStruct((n, _DIM), jnp.float32),
            jax.ShapeDtypeStruct((n, 1), jnp.float32),
        ],
        compiler_params=pltpu.CompilerParams(
            dimension_semantics=("parallel",),
        ),
    )(samples, wa, wb, ba, bb)
    return (res, jac.reshape(n))


# speedup vs baseline: 1.2118x; 1.2118x over previous
import jax
import jax.numpy as jnp
from jax.experimental import pallas as pl
from jax.experimental.pallas import tpu as pltpu

_DIM = 128
_HALF = 64
_BLOCK = 1024


def _coupling_kernel(x_ref, wa_ref, wb_ref, ba_ref, bb_ref, out_ref, jac_ref):
    x = x_ref[:, :]
    u = jnp.dot(x, wa_ref[:, :], preferred_element_type=jnp.float32) + ba_ref[0, :]
    sf = jnp.tanh(u)
    t2 = jnp.dot(x, wb_ref[:, :], preferred_element_type=jnp.float32) + bb_ref[0, :]
    out_ref[:, :] = x * jnp.exp(sf) + t2
    ones_row = jnp.ones((1, _DIM), dtype=jnp.float32)
    jac_ref[0, :, :] = jax.lax.dot_general(
        ones_row, sf, (((1,), (1,)), ((), ())),
        precision=jax.lax.Precision.HIGHEST,
        preferred_element_type=jnp.float32)


def kernel(samples, W_s, b_s, W_t, b_t):
    n = samples.shape[0]
    zpad = jnp.zeros((_DIM, _DIM), dtype=jnp.float32)
    wa = zpad.at[_HALF:, :_HALF].set(W_s)
    wb = zpad.at[_HALF:, :_HALF].set(W_t)
    zvec = jnp.zeros((_HALF,), dtype=jnp.float32)
    ba = jnp.concatenate([b_s, zvec]).reshape(1, _DIM)
    bb = jnp.concatenate([b_t, zvec]).reshape(1, _DIM)
    grid = n // _BLOCK
    res, jac = pl.pallas_call(
        _coupling_kernel,
        grid=(grid,),
        in_specs=[
            pl.BlockSpec((_BLOCK, _DIM), lambda i: (i, 0)),
            pl.BlockSpec((_DIM, _DIM), lambda i: (0, 0)),
            pl.BlockSpec((_DIM, _DIM), lambda i: (0, 0)),
            pl.BlockSpec((1, _DIM), lambda i: (0, 0)),
            pl.BlockSpec((1, _DIM), lambda i: (0, 0)),
        ],
        out_specs=[
            pl.BlockSpec((_BLOCK, _DIM), lambda i: (i, 0)),
            pl.BlockSpec((1, 1, _BLOCK), lambda i: (i, 0, 0)),
        ],
        out_shape=[
            jax.ShapeDtypeStruct((n, _DIM), jnp.float32),
            jax.ShapeDtypeStruct((n // _BLOCK, 1, _BLOCK), jnp.float32),
        ],
        compiler_params=pltpu.CompilerParams(
            dimension_semantics=("parallel",),
        ),
    )(samples, wa, wb, ba, bb)
    return (res, jac.reshape(n))


# speedup vs baseline: 1.8063x; 1.4905x over previous
import jax
import jax.numpy as jnp
from jax.experimental import pallas as pl
from jax.experimental.pallas import tpu as pltpu

_DIM = 128
_HALF = 64
_BLOCK = 2048


def _coupling_kernel(x_ref, wa_ref, wb_ref, ba_ref, bb_ref, out_ref, jac_ref):
    x = x_ref[:, :]
    u = jnp.dot(x, wa_ref[:, :], preferred_element_type=jnp.float32) + ba_ref[0, :]
    sf = jnp.tanh(u)
    t2 = jnp.dot(x, wb_ref[:, :], preferred_element_type=jnp.float32) + bb_ref[0, :]
    out_ref[:, :] = x * jnp.exp(sf) + t2
    ones_row = jnp.ones((1, _DIM), dtype=jnp.float32)
    jac_ref[0, :, :] = jax.lax.dot_general(
        ones_row, sf, (((1,), (1,)), ((), ())),
        precision=jax.lax.Precision.HIGHEST,
        preferred_element_type=jnp.float32)


def kernel(samples, W_s, b_s, W_t, b_t):
    n = samples.shape[0]
    zpad = jnp.zeros((_DIM, _DIM), dtype=jnp.float32)
    wa = zpad.at[_HALF:, :_HALF].set(W_s)
    wb = zpad.at[_HALF:, :_HALF].set(W_t)
    zvec = jnp.zeros((_HALF,), dtype=jnp.float32)
    ba = jnp.concatenate([b_s, zvec]).reshape(1, _DIM)
    bb = jnp.concatenate([b_t, zvec]).reshape(1, _DIM)
    grid = n // _BLOCK
    res, jac = pl.pallas_call(
        _coupling_kernel,
        grid=(grid,),
        in_specs=[
            pl.BlockSpec((_BLOCK, _DIM), lambda i: (i, 0)),
            pl.BlockSpec((_DIM, _DIM), lambda i: (0, 0)),
            pl.BlockSpec((_DIM, _DIM), lambda i: (0, 0)),
            pl.BlockSpec((1, _DIM), lambda i: (0, 0)),
            pl.BlockSpec((1, _DIM), lambda i: (0, 0)),
        ],
        out_specs=[
            pl.BlockSpec((_BLOCK, _DIM), lambda i: (i, 0)),
            pl.BlockSpec((1, 1, _BLOCK), lambda i: (i, 0, 0)),
        ],
        out_shape=[
            jax.ShapeDtypeStruct((n, _DIM), jnp.float32),
            jax.ShapeDtypeStruct((n // _BLOCK, 1, _BLOCK), jnp.float32),
        ],
        compiler_params=pltpu.CompilerParams(
            dimension_semantics=("parallel",),
        ),
    )(samples, wa, wb, ba, bb)
    return (res, jac.reshape(n))


# speedup vs baseline: 1.9780x; 1.0951x over previous
import jax
import jax.numpy as jnp
from jax.experimental import pallas as pl
from jax.experimental.pallas import tpu as pltpu

_DIM = 128
_HALF = 64
_BLOCK = 4096


def _coupling_kernel(x_ref, wa_ref, wb_ref, ba_ref, bb_ref, out_ref, jac_ref):
    x = x_ref[:, :]
    u = jnp.dot(x, wa_ref[:, :], preferred_element_type=jnp.float32) + ba_ref[0, :]
    sf = jnp.tanh(u)
    t2 = jnp.dot(x, wb_ref[:, :], preferred_element_type=jnp.float32) + bb_ref[0, :]
    out_ref[:, :] = x * jnp.exp(sf) + t2
    ones_row = jnp.ones((1, _DIM), dtype=jnp.float32)
    jac_ref[0, :, :] = jax.lax.dot_general(
        ones_row, sf, (((1,), (1,)), ((), ())),
        precision=jax.lax.Precision.HIGHEST,
        preferred_element_type=jnp.float32)


def kernel(samples, W_s, b_s, W_t, b_t):
    n = samples.shape[0]
    zpad = jnp.zeros((_DIM, _DIM), dtype=jnp.float32)
    wa = zpad.at[_HALF:, :_HALF].set(W_s)
    wb = zpad.at[_HALF:, :_HALF].set(W_t)
    zvec = jnp.zeros((_HALF,), dtype=jnp.float32)
    ba = jnp.concatenate([b_s, zvec]).reshape(1, _DIM)
    bb = jnp.concatenate([b_t, zvec]).reshape(1, _DIM)
    grid = n // _BLOCK
    res, jac = pl.pallas_call(
        _coupling_kernel,
        grid=(grid,),
        in_specs=[
            pl.BlockSpec((_BLOCK, _DIM), lambda i: (i, 0)),
            pl.BlockSpec((_DIM, _DIM), lambda i: (0, 0)),
            pl.BlockSpec((_DIM, _DIM), lambda i: (0, 0)),
            pl.BlockSpec((1, _DIM), lambda i: (0, 0)),
            pl.BlockSpec((1, _DIM), lambda i: (0, 0)),
        ],
        out_specs=[
            pl.BlockSpec((_BLOCK, _DIM), lambda i: (i, 0)),
            pl.BlockSpec((1, 1, _BLOCK), lambda i: (i, 0, 0)),
        ],
        out_shape=[
            jax.ShapeDtypeStruct((n, _DIM), jnp.float32),
            jax.ShapeDtypeStruct((n // _BLOCK, 1, _BLOCK), jnp.float32),
        ],
        compiler_params=pltpu.CompilerParams(
            dimension_semantics=("parallel",),
        ),
    )(samples, wa, wb, ba, bb)
    return (res, jac.reshape(n))


# speedup vs baseline: 2.0340x; 1.0283x over previous
import jax
import jax.numpy as jnp
from jax.experimental import pallas as pl
from jax.experimental.pallas import tpu as pltpu

_DIM = 128
_HALF = 64
_BLOCK = 8192


def _coupling_kernel(x_ref, wa_ref, wb_ref, ba_ref, bb_ref, out_ref, jac_ref):
    x = x_ref[:, :]
    u = jnp.dot(x, wa_ref[:, :], preferred_element_type=jnp.float32) + ba_ref[0, :]
    sf = jnp.tanh(u)
    t2 = jnp.dot(x, wb_ref[:, :], preferred_element_type=jnp.float32) + bb_ref[0, :]
    out_ref[:, :] = x * jnp.exp(sf) + t2
    ones_row = jnp.ones((1, _DIM), dtype=jnp.float32)
    jac_ref[0, :, :] = jax.lax.dot_general(
        ones_row, sf, (((1,), (1,)), ((), ())),
        precision=jax.lax.Precision.HIGHEST,
        preferred_element_type=jnp.float32)


def kernel(samples, W_s, b_s, W_t, b_t):
    n = samples.shape[0]
    zpad = jnp.zeros((_DIM, _DIM), dtype=jnp.float32)
    wa = zpad.at[_HALF:, :_HALF].set(W_s)
    wb = zpad.at[_HALF:, :_HALF].set(W_t)
    zvec = jnp.zeros((_HALF,), dtype=jnp.float32)
    ba = jnp.concatenate([b_s, zvec]).reshape(1, _DIM)
    bb = jnp.concatenate([b_t, zvec]).reshape(1, _DIM)
    grid = n // _BLOCK
    res, jac = pl.pallas_call(
        _coupling_kernel,
        grid=(grid,),
        in_specs=[
            pl.BlockSpec((_BLOCK, _DIM), lambda i: (i, 0)),
            pl.BlockSpec((_DIM, _DIM), lambda i: (0, 0)),
            pl.BlockSpec((_DIM, _DIM), lambda i: (0, 0)),
            pl.BlockSpec((1, _DIM), lambda i: (0, 0)),
            pl.BlockSpec((1, _DIM), lambda i: (0, 0)),
        ],
        out_specs=[
            pl.BlockSpec((_BLOCK, _DIM), lambda i: (i, 0)),
            pl.BlockSpec((1, 1, _BLOCK), lambda i: (i, 0, 0)),
        ],
        out_shape=[
            jax.ShapeDtypeStruct((n, _DIM), jnp.float32),
            jax.ShapeDtypeStruct((n // _BLOCK, 1, _BLOCK), jnp.float32),
        ],
        compiler_params=pltpu.CompilerParams(
            dimension_semantics=("parallel",),
        ),
    )(samples, wa, wb, ba, bb)
    return (res, jac.reshape(n))


# speedup vs baseline: 2.0356x; 1.0008x over previous
import jax
import jax.numpy as jnp
from jax.experimental import pallas as pl
from jax.experimental.pallas import tpu as pltpu

_DIM = 128
_HALF = 64
_BLOCK = 16384


def _coupling_kernel(x_ref, wa_ref, wb_ref, ba_ref, bb_ref, out_ref, jac_ref):
    x = x_ref[:, :]
    u = jnp.dot(x, wa_ref[:, :], preferred_element_type=jnp.float32) + ba_ref[0, :]
    sf = jnp.tanh(u)
    t2 = jnp.dot(x, wb_ref[:, :], preferred_element_type=jnp.float32) + bb_ref[0, :]
    out_ref[:, :] = x * jnp.exp(sf) + t2
    ones_row = jnp.ones((1, _DIM), dtype=jnp.float32)
    jac_ref[0, :, :] = jax.lax.dot_general(
        ones_row, sf, (((1,), (1,)), ((), ())),
        precision=jax.lax.Precision.HIGHEST,
        preferred_element_type=jnp.float32)


def kernel(samples, W_s, b_s, W_t, b_t):
    n = samples.shape[0]
    zpad = jnp.zeros((_DIM, _DIM), dtype=jnp.float32)
    wa = zpad.at[_HALF:, :_HALF].set(W_s)
    wb = zpad.at[_HALF:, :_HALF].set(W_t)
    zvec = jnp.zeros((_HALF,), dtype=jnp.float32)
    ba = jnp.concatenate([b_s, zvec]).reshape(1, _DIM)
    bb = jnp.concatenate([b_t, zvec]).reshape(1, _DIM)
    grid = n // _BLOCK
    res, jac = pl.pallas_call(
        _coupling_kernel,
        grid=(grid,),
        in_specs=[
            pl.BlockSpec((_BLOCK, _DIM), lambda i: (i, 0)),
            pl.BlockSpec((_DIM, _DIM), lambda i: (0, 0)),
            pl.BlockSpec((_DIM, _DIM), lambda i: (0, 0)),
            pl.BlockSpec((1, _DIM), lambda i: (0, 0)),
            pl.BlockSpec((1, _DIM), lambda i: (0, 0)),
        ],
        out_specs=[
            pl.BlockSpec((_BLOCK, _DIM), lambda i: (i, 0)),
            pl.BlockSpec((1, 1, _BLOCK), lambda i: (i, 0, 0)),
        ],
        out_shape=[
            jax.ShapeDtypeStruct((n, _DIM), jnp.float32),
            jax.ShapeDtypeStruct((n // _BLOCK, 1, _BLOCK), jnp.float32),
        ],
        compiler_params=pltpu.CompilerParams(
            dimension_semantics=("parallel",),
        ),
    )(samples, wa, wb, ba, bb)
    return (res, jac.reshape(n))


# speedup vs baseline: 3.3256x; 1.6337x over previous
import jax
import jax.numpy as jnp
from jax.experimental import pallas as pl
from jax.experimental.pallas import tpu as pltpu

_DIM = 128
_HALF = 64
_BLOCK = 16384


def _coupling_kernel(x_ref, wa_ref, wb_ref, ba_ref, bb_ref, out_ref, jac_ref):
    x = x_ref[:, :]
    u = jnp.dot(x, wa_ref[:, :], preferred_element_type=jnp.float32) + ba_ref[0, :]
    sf = jnp.tanh(u)
    t2 = jnp.dot(x, wb_ref[:, :], preferred_element_type=jnp.float32) + bb_ref[0, :]
    out_ref[:, :] = x * jnp.exp(sf) + t2
    ones_row = jnp.ones((1, _DIM), dtype=jnp.float32)
    sf_hi = sf.astype(jnp.bfloat16).astype(jnp.float32)
    sf_lo = sf - sf_hi
    dims = (((1,), (1,)), ((), ()))
    jac_ref[0, :, :] = (
        jax.lax.dot_general(ones_row, sf_hi, dims,
                            preferred_element_type=jnp.float32)
        + jax.lax.dot_general(ones_row, sf_lo, dims,
                              preferred_element_type=jnp.float32))


def kernel(samples, W_s, b_s, W_t, b_t):
    n = samples.shape[0]
    zpad = jnp.zeros((_DIM, _DIM), dtype=jnp.float32)
    wa = zpad.at[_HALF:, :_HALF].set(W_s)
    wb = zpad.at[_HALF:, :_HALF].set(W_t)
    zvec = jnp.zeros((_HALF,), dtype=jnp.float32)
    ba = jnp.concatenate([b_s, zvec]).reshape(1, _DIM)
    bb = jnp.concatenate([b_t, zvec]).reshape(1, _DIM)
    grid = n // _BLOCK
    res, jac = pl.pallas_call(
        _coupling_kernel,
        grid=(grid,),
        in_specs=[
            pl.BlockSpec((_BLOCK, _DIM), lambda i: (i, 0)),
            pl.BlockSpec((_DIM, _DIM), lambda i: (0, 0)),
            pl.BlockSpec((_DIM, _DIM), lambda i: (0, 0)),
            pl.BlockSpec((1, _DIM), lambda i: (0, 0)),
            pl.BlockSpec((1, _DIM), lambda i: (0, 0)),
        ],
        out_specs=[
            pl.BlockSpec((_BLOCK, _DIM), lambda i: (i, 0)),
            pl.BlockSpec((1, 1, _BLOCK), lambda i: (i, 0, 0)),
        ],
        out_shape=[
            jax.ShapeDtypeStruct((n, _DIM), jnp.float32),
            jax.ShapeDtypeStruct((n // _BLOCK, 1, _BLOCK), jnp.float32),
        ],
        compiler_params=pltpu.CompilerParams(
            dimension_semantics=("parallel",),
        ),
    )(samples, wa, wb, ba, bb)
    return (res, jac.reshape(n))
